# baseline (speedup 1.0000x reference)
import functools

import jax
import jax.numpy as jnp
from jax import lax
from jax.experimental import pallas as pl
from jax.experimental.pallas import tpu as pltpu
from jax.experimental.pallas import tpu_sc as plsc

_NUM_CORES = 2
_NUM_SUBCORES = 16
_NW = _NUM_CORES * _NUM_SUBCORES
_LANES = 16
_NBUF = 4


def _sc_pool_fc(seq_r, lengths, fc_w, fc_b_pad, emb_table, L, B, D):
    BPW = B // _NW
    n_chunks = D // _LANES

    mesh = plsc.VectorSubcoreMesh(core_axis_name="c", subcore_axis_name="s")

    @functools.partial(
        pl.kernel,
        mesh=mesh,
        out_type=jax.ShapeDtypeStruct((B, _LANES), jnp.float32),
        compiler_params=pltpu.CompilerParams(
            needs_layout_passes=False, use_tc_tiling_on_sc=False),
        scratch_types=(
            [pltpu.VMEM((BPW * L,), jnp.int32)]
            + [pltpu.VMEM((L, D), jnp.float32) for _ in range(_NBUF)]
            + [pltpu.VMEM((BPW,), jnp.int32)]
            + [pltpu.VMEM((BPW,), jnp.float32)]
            + [pltpu.VMEM((2, D), jnp.float32)]
            + [pltpu.VMEM((_LANES,), jnp.float32)]
            + [pltpu.VMEM((BPW, _LANES), jnp.float32)]
            + [pltpu.SemaphoreType.DMA for _ in range(_NBUF)]
        ),
    )
    def body(seq_hbm, len_hbm, w_hbm, b_hbm, table_hbm, out_hbm,
             idx_v, r0, r1, r2, r3, len_v, invlen_v, w_v, b_v, out_v,
             s0, s1, s2, s3):
        rows = (r0, r1, r2, r3)
        sems = (s0, s1, s2, s3)
        wid = lax.axis_index("s") * _NUM_CORES + lax.axis_index("c")
        base = wid * BPW

        pltpu.sync_copy(seq_hbm.at[wid], idx_v)
        pltpu.sync_copy(len_hbm.at[pl.ds(base, BPW)], len_v)
        pltpu.sync_copy(w_hbm, w_v)
        pltpu.sync_copy(b_hbm, b_v)

        def start_gather(g, buf_i):
            pltpu.make_async_copy(
                table_hbm.at[idx_v.at[pl.ds(g * L, L)]], rows[buf_i], sems[buf_i]
            ).start()

        def wait_gather(g, buf_i):
            pltpu.make_async_copy(
                table_hbm.at[idx_v.at[pl.ds(g * L, L)]], rows[buf_i], sems[buf_i]
            ).wait()

        w_chunks = [
            (w_v[0, pl.ds(c * _LANES, _LANES)], w_v[1, pl.ds(c * _LANES, _LANES)])
            for c in range(n_chunks)
        ]
        bias_vec = b_v[:]
        lane = jnp.arange(_LANES, dtype=jnp.int32)
        zero = jnp.zeros((_LANES,), jnp.float32)

        for k in range(BPW // _LANES):
            lv = len_v[pl.ds(k * _LANES, _LANES)].astype(jnp.float32)
            invlen_v[pl.ds(k * _LANES, _LANES)] = 1.0 / lv

        for b in range(_NBUF):
            start_gather(b, b)

        def compute(g, buf, inv):
            def tbody(i, accs):
                a = accs
                t = i * 2
                for dt in range(2):
                    r = t + dt
                    a = tuple(
                        a[c] + buf[r, pl.ds(c * _LANES, _LANES)]
                        for c in range(n_chunks)
                    )
                return a

            accs = lax.fori_loop(
                0, L // 2, tbody, tuple(zero for _ in range(n_chunks))
            )
            p0 = zero
            p1 = zero
            for c in range(n_chunks):
                p0 = p0 + accs[c] * w_chunks[c][0]
                p1 = p1 + accs[c] * w_chunks[c][1]
            s0 = plsc.cumsum(p0)[_LANES - 1]
            s1 = plsc.cumsum(p1)[_LANES - 1]
            sel = jnp.where(lane == 0, s0, jnp.where(lane == 1, s1, 0.0))
            vec = sel * inv + bias_vec
            out_v[g, :] = 1.0 / (1.0 + jnp.exp(-vec))

        def outer(j, carry):
            iv = invlen_v[pl.ds(j * _LANES, _LANES)]
            for b in range(_LANES):
                g = j * _LANES + b
                buf_i = b % _NBUF
                wait_gather(g, buf_i)
                compute(g, rows[buf_i], jnp.full((_LANES,), iv[b]))
                nxt = g + _NBUF

                @pl.when(nxt < BPW)
                def _():
                    start_gather(nxt, buf_i)
            return carry

        lax.fori_loop(0, BPW // _LANES, outer, 0)
        pltpu.sync_copy(out_v, out_hbm.at[pl.ds(base, BPW)])

    return body(seq_r, lengths, fc_w, fc_b_pad, emb_table)


def kernel(seq, lengths, emb_table, fc_w, fc_b):
    L, B = seq.shape
    V, D = emb_table.shape
    seq_r = seq.T.reshape(_NW, (B // _NW) * L).astype(jnp.int32)
    fc_b_pad = jnp.zeros((_LANES,), jnp.float32).at[:2].set(fc_b)
    out16 = _sc_pool_fc(seq_r, lengths.astype(jnp.int32), fc_w, fc_b_pad,
                        emb_table, L, B, D)
    return out16[:, :2]
